# baseline (speedup 1.0000x reference)
import functools

import jax
import jax.numpy as jnp
from jax import lax
from jax.experimental import pallas as pl
from jax.experimental.pallas import tpu as pltpu
from jax.experimental.pallas import tpu_sc as plsc

N = 10000
E = 320000
D_IN = 128
D_H = 256
D_OUT = 64

NC = 2
NS = 16
NW = NC * NS

B = 128
ITERS = 81
NG = ITERS // 3
E_PAD = NW * ITERS * B
RPT = 632
N_PAD = NS * RPT
DUMMY = N
SRPT = 625

CA = 64
CD = 16

_SC_PARAMS = pltpu.CompilerParams(use_tc_tiling_on_sc=False)
_MESH = dict(core_axis_name="c", subcore_axis_name="s")


def _sc_agg_body(K, xr, edges, zeros, out, *scratch):
    e_srcv, e_dstv = scratch[0], scratch[1]
    rows = scratch[2:5]
    aggS, xS = scratch[5], scratch[6]
    sg = scratch[7:10]
    ss = scratch[10:13]
    c = lax.axis_index("c")
    s = lax.axis_index("s")
    wid = c * NS + s
    r0 = s * RPT
    x0 = s * SRPT
    pltpu.sync_copy(edges.at[0, wid], e_srcv)
    pltpu.sync_copy(edges.at[1, wid], e_dstv)
    for k in range(K):
        pltpu.sync_copy(zeros.at[pl.ds(r0, RPT)], aggS.at[pl.ds(r0, RPT)])
        pltpu.sync_copy(xr.at[pl.ds(x0, SRPT), k], xS.at[pl.ds(x0, SRPT)])
        plsc.subcore_barrier()

        def fire_g(j, b):
            pltpu.async_copy(xS.at[e_srcv.at[j]], rows[b], sg[b])

        def wait_g(j, b):
            pltpu.make_async_copy(xS.at[e_srcv.at[j]], rows[b], sg[b]).wait()

        def fire_s(j, b):
            pltpu.async_copy(rows[b], aggS.at[e_dstv.at[j]], ss[b], add=True)

        def wait_s(j, b):
            pltpu.make_async_copy(rows[b], aggS.at[e_dstv.at[j]], ss[b]).wait()

        fire_g(0, 0)
        fire_g(1, 1)

        def it(i, carry):
            for r in range(3):
                j = 3 * i + r
                jm = j - 1
                if r == 0:
                    @pl.when(i >= 1)
                    def _():
                        wait_s(jm, 2)
                else:
                    wait_s(jm, (r - 1) % 3)
                jn = j + 2
                if r == 0:
                    fire_g(jn, 2)
                else:
                    @pl.when(i < NG - 1)
                    def _():
                        fire_g(jn, (r + 2) % 3)
                wait_g(j, r)
                fire_s(j, r)
            return carry

        lax.fori_loop(0, NG, it, 0)
        wait_s(ITERS - 1, 2)
        plsc.subcore_barrier()
        pltpu.sync_copy(aggS.at[pl.ds(r0, RPT)], out.at[c, k, pl.ds(r0, RPT)])
        if k + 1 < K:
            plsc.subcore_barrier()


@functools.cache
def _make_sc_agg(K):
    return pl.kernel(
        functools.partial(_sc_agg_body, K),
        out_type=jax.ShapeDtypeStruct((NC, K, N_PAD, CA), jnp.float32),
        mesh=plsc.VectorSubcoreMesh(**_MESH),
        scratch_types=[
            pltpu.VMEM((ITERS, B), jnp.int32),
            pltpu.VMEM((ITERS, B), jnp.int32),
            pltpu.VMEM((B, CA), jnp.float32),
            pltpu.VMEM((B, CA), jnp.float32),
            pltpu.VMEM((B, CA), jnp.float32),
            pltpu.VMEM_SHARED((N_PAD, CA), jnp.float32),
            pltpu.VMEM_SHARED((N, CA), jnp.float32),
            pltpu.SemaphoreType.DMA,
            pltpu.SemaphoreType.DMA,
            pltpu.SemaphoreType.DMA,
            pltpu.SemaphoreType.DMA,
            pltpu.SemaphoreType.DMA,
            pltpu.SemaphoreType.DMA,
        ],
        compiler_params=_SC_PARAMS,
    )


def _sc_agg(xr, edges):
    K = xr.shape[1]
    zeros = jnp.zeros((N_PAD, CA), jnp.float32)
    return _make_sc_agg(K)(xr, edges, zeros)


def _sc_deg_body(edges, ones_h, zeros_d, out_d, e_dstv, ones_v, degS, ss0, ss1):
    c = lax.axis_index("c")
    s = lax.axis_index("s")
    wid = c * NS + s
    r0 = s * RPT
    pltpu.sync_copy(edges.at[1, wid], e_dstv)
    pltpu.sync_copy(ones_h, ones_v)
    pltpu.sync_copy(zeros_d.at[pl.ds(r0, RPT)], degS.at[pl.ds(r0, RPT)])
    plsc.subcore_barrier()

    def fire_s(j, sem):
        pltpu.async_copy(ones_v, degS.at[e_dstv.at[j]], sem, add=True)

    def wait_s(j, sem):
        pltpu.make_async_copy(ones_v, degS.at[e_dstv.at[j]], sem).wait()

    def it(i, carry):
        j = 3 * i
        fire_s(j, ss0)
        fire_s(j + 1, ss1)
        fire_s(j + 2, ss0)
        wait_s(j, ss0)
        wait_s(j + 1, ss1)
        wait_s(j + 2, ss0)
        return carry

    lax.fori_loop(0, NG, it, 0)
    plsc.subcore_barrier()
    pltpu.sync_copy(degS.at[pl.ds(r0, RPT)], out_d.at[c, pl.ds(r0, RPT)])


@functools.cache
def _make_sc_deg():
    return pl.kernel(
        _sc_deg_body,
        out_type=jax.ShapeDtypeStruct((NC, N_PAD, CD), jnp.float32),
        mesh=plsc.VectorSubcoreMesh(**_MESH),
        scratch_types=[
            pltpu.VMEM((ITERS, B), jnp.int32),
            pltpu.VMEM((B, CD), jnp.float32),
            pltpu.VMEM_SHARED((N_PAD, CD), jnp.float32),
            pltpu.SemaphoreType.DMA,
            pltpu.SemaphoreType.DMA,
        ],
        compiler_params=_SC_PARAMS,
    )


def _deg_of(pd_ref):
    return jnp.maximum(pd_ref[0][:, 0:1] + pd_ref[1][:, 0:1], 1.0)


def _mm0_body(x_ref, p_ref, pd_ref, ws_ref, wn_ref, b_ref, h_ref):
    hn = jnp.concatenate(
        [p_ref[0, 0] + p_ref[1, 0], p_ref[0, 1] + p_ref[1, 1]],
        axis=1) / _deg_of(pd_ref)
    h = (jnp.dot(x_ref[...], ws_ref[...], preferred_element_type=jnp.float32)
         + jnp.dot(hn, wn_ref[...], preferred_element_type=jnp.float32)
         + b_ref[...])
    h_ref[...] = jnp.maximum(h, 0.0)


def _mm1_body(h0_ref, p1_ref, pd_ref, ws_ref, wn_ref, b_ref, wn2_ref,
              h1_ref, y2_ref):
    hn = jnp.concatenate(
        [p1_ref[0, j] + p1_ref[1, j] for j in range(4)],
        axis=1) / _deg_of(pd_ref)
    h1 = (jnp.dot(h0_ref[...], ws_ref[...], preferred_element_type=jnp.float32)
          + jnp.dot(hn, wn_ref[...], preferred_element_type=jnp.float32)
          + b_ref[...])
    h1 = jnp.maximum(h1, 0.0)
    h1_ref[...] = h1
    y2_ref[...] = jnp.dot(h1, wn2_ref[...], preferred_element_type=jnp.float32)


def _mm2_body(h1_ref, p2_ref, pd_ref, ws_ref, b_ref, o_ref):
    hn = (p2_ref[0, 0] + p2_ref[1, 0]) / _deg_of(pd_ref)
    o_ref[...] = (jnp.dot(h1_ref[...], ws_ref[...],
                          preferred_element_type=jnp.float32)
                  + hn + b_ref[...])


BN = 1000
_G = N // BN


def _full(shape):
    return pl.BlockSpec(shape, lambda i: tuple(0 for _ in shape))


def _rows(shape):
    nd = len(shape)
    return pl.BlockSpec(shape, lambda i, nd=nd: tuple(
        i if d == nd - 2 else 0 for d in range(nd)))


def kernel(inputs, edge_index, W_self0, W_neigh0, b0, W_self1, W_neigh1, b1,
           W_self2, W_neigh2, b2):
    x = inputs
    pad = E_PAD - E
    padv = jnp.concatenate(
        [jnp.zeros((1, pad), jnp.int32), jnp.full((1, pad), DUMMY, jnp.int32)])
    edges = jnp.concatenate([edge_index, padv], axis=1).reshape(2, NW, ITERS, B)

    pd = _make_sc_deg()(edges, jnp.ones((B, CD), jnp.float32),
                        jnp.zeros((N_PAD, CD), jnp.float32))

    p0 = _sc_agg(x.reshape(N, 2, CA), edges)

    h0 = pl.pallas_call(
        _mm0_body,
        grid=(_G,),
        in_specs=[
            _rows((BN, D_IN)),
            _rows((2, 2, BN, CA)),
            _rows((2, BN, CD)),
            _full((D_IN, D_H)),
            _full((D_IN, D_H)),
            _full((1, D_H)),
        ],
        out_specs=_rows((BN, D_H)),
        out_shape=jax.ShapeDtypeStruct((N, D_H), jnp.float32),
    )(x, p0, pd, W_self0, W_neigh0, b0.reshape(1, -1))

    p1 = _sc_agg(h0.reshape(N, 4, CA), edges)

    h1, y2 = pl.pallas_call(
        _mm1_body,
        grid=(_G,),
        in_specs=[
            _rows((BN, D_H)),
            _rows((2, 4, BN, CA)),
            _rows((2, BN, CD)),
            _full((D_H, D_H)),
            _full((D_H, D_H)),
            _full((1, D_H)),
            _full((D_H, D_OUT)),
        ],
        out_specs=[_rows((BN, D_H)), _rows((BN, D_OUT))],
        out_shape=[jax.ShapeDtypeStruct((N, D_H), jnp.float32),
                   jax.ShapeDtypeStruct((N, D_OUT), jnp.float32)],
    )(h0, p1, pd, W_self1, W_neigh1, b1.reshape(1, -1), W_neigh2)

    p2 = _sc_agg(y2.reshape(N, 1, CA), edges)

    out = pl.pallas_call(
        _mm2_body,
        grid=(_G,),
        in_specs=[
            _rows((BN, D_H)),
            _rows((2, 1, BN, CA)),
            _rows((2, BN, CD)),
            _full((D_H, D_OUT)),
            _full((1, D_OUT)),
        ],
        out_specs=_rows((BN, D_OUT)),
        out_shape=jax.ShapeDtypeStruct((N, D_OUT), jnp.float32),
    )(h1, p2, pd, W_self2, b2.reshape(1, -1))

    return (out, h0, h1)


# speedup vs baseline: 1.3360x; 1.3360x over previous
import functools

import jax
import jax.numpy as jnp
from jax import lax
from jax.experimental import pallas as pl
from jax.experimental.pallas import tpu as pltpu
from jax.experimental.pallas import tpu_sc as plsc

N = 10000
E = 320000
D_IN = 128
D_H = 256
D_OUT = 64

NC = 2
NS = 16
NW = NC * NS

B = 80
ITERS = E // (NW * B)
NR = 5
NG = ITERS // NR
RPT = N // NS

CA = 64
CD = 16

_SC_PARAMS = pltpu.CompilerParams(use_tc_tiling_on_sc=False)
_MESH = dict(core_axis_name="c", subcore_axis_name="s")


def _sc_agg_body(K, *args):
    xs = args[:K]
    edges, zeros, out = args[K:K + 3]
    scratch = args[K + 3:]
    e_srcv, e_dstv = scratch[0], scratch[1]
    rows = scratch[2:2 + NR]
    aggS, xS = scratch[2 + NR], scratch[3 + NR]
    sg = scratch[4 + NR:4 + 2 * NR]
    ss = scratch[4 + 2 * NR:4 + 3 * NR]
    c = lax.axis_index("c")
    s = lax.axis_index("s")
    wid = c * NS + s
    r0 = s * RPT
    pltpu.sync_copy(edges.at[0, pl.ds(wid * ITERS, ITERS)], e_srcv)
    pltpu.sync_copy(edges.at[1, pl.ds(wid * ITERS, ITERS)], e_dstv)
    for k in range(K):
        xk = xs[k]
        pltpu.sync_copy(zeros.at[pl.ds(r0, RPT)], aggS.at[pl.ds(r0, RPT)])
        pltpu.sync_copy(xk.at[pl.ds(r0, RPT)], xS.at[pl.ds(r0, RPT)])
        plsc.subcore_barrier()

        def fire_g(j, b, xk=xk):
            pltpu.async_copy(xS.at[e_srcv.at[j]], rows[b], sg[b])

        def wait_g(j, b, xk=xk):
            pltpu.make_async_copy(xS.at[e_srcv.at[j]], rows[b], sg[b]).wait()

        def fire_s(j, b):
            pltpu.async_copy(rows[b], aggS.at[e_dstv.at[j]], ss[b], add=True)

        def wait_s(j, b):
            pltpu.make_async_copy(rows[b], aggS.at[e_dstv.at[j]], ss[b]).wait()

        fire_g(0, 0)
        fire_g(1, 1)
        fire_g(2, 2)

        def it(i, carry):
            for r in range(NR):
                j = NR * i + r
                b3 = (r + 3) % NR
                if r <= 1:
                    @pl.when(i >= 1)
                    def _():
                        wait_s(j - 2, b3)

                    fire_g(j + 3, b3)
                else:
                    wait_s(j - 2, b3)

                    @pl.when(i < NG - 1)
                    def _():
                        fire_g(j + 3, b3)
                wait_g(j, r)
                fire_s(j, r)
            return carry

        lax.fori_loop(0, NG, it, 0)
        wait_s(ITERS - 2, (ITERS - 2) % NR)
        wait_s(ITERS - 1, (ITERS - 1) % NR)
        plsc.subcore_barrier()
        pltpu.sync_copy(aggS.at[pl.ds(r0, RPT)], out.at[c, k, pl.ds(r0, RPT)])
        if k + 1 < K:
            plsc.subcore_barrier()


@functools.cache
def _make_sc_agg(K):
    return pl.kernel(
        functools.partial(_sc_agg_body, K),
        out_type=jax.ShapeDtypeStruct((NC, K, N, CA), jnp.float32),
        mesh=plsc.VectorSubcoreMesh(**_MESH),
        scratch_types=(
            [pltpu.VMEM((ITERS, B), jnp.int32)] * 2
            + [pltpu.VMEM((B, CA), jnp.float32)] * NR
            + [pltpu.VMEM_SHARED((N, CA), jnp.float32)] * 2
            + [pltpu.SemaphoreType.DMA] * (2 * NR)
        ),
        compiler_params=_SC_PARAMS,
    )


def _sc_agg(xchunks, edges):
    zeros = jnp.zeros((N, CA), jnp.float32)
    return _make_sc_agg(len(xchunks))(*xchunks, edges, zeros)


def _sc_deg_body(edges, ones_h, zeros_d, out_d, e_dstv, ones_v, degS, ss0, ss1):
    c = lax.axis_index("c")
    s = lax.axis_index("s")
    wid = c * NS + s
    r0 = s * RPT
    pltpu.sync_copy(edges.at[1, pl.ds(wid * ITERS, ITERS)], e_dstv)
    pltpu.sync_copy(ones_h, ones_v)
    pltpu.sync_copy(zeros_d.at[pl.ds(r0, RPT)], degS.at[pl.ds(r0, RPT)])
    plsc.subcore_barrier()

    def fire_s(j, sem):
        pltpu.async_copy(ones_v, degS.at[e_dstv.at[j]], sem, add=True)

    def wait_s(j, sem):
        pltpu.make_async_copy(ones_v, degS.at[e_dstv.at[j]], sem).wait()

    def it(i, carry):
        j = NR * i
        for r in range(NR):
            fire_s(j + r, ss0 if r % 2 == 0 else ss1)
        for r in range(NR):
            wait_s(j + r, ss0 if r % 2 == 0 else ss1)
        return carry

    lax.fori_loop(0, NG, it, 0)
    plsc.subcore_barrier()
    pltpu.sync_copy(degS.at[pl.ds(r0, RPT)], out_d.at[c, pl.ds(r0, RPT)])


@functools.cache
def _make_sc_deg():
    return pl.kernel(
        _sc_deg_body,
        out_type=jax.ShapeDtypeStruct((NC, N, CD), jnp.float32),
        mesh=plsc.VectorSubcoreMesh(**_MESH),
        scratch_types=[
            pltpu.VMEM((ITERS, B), jnp.int32),
            pltpu.VMEM((B, CD), jnp.float32),
            pltpu.VMEM_SHARED((N, CD), jnp.float32),
            pltpu.SemaphoreType.DMA,
            pltpu.SemaphoreType.DMA,
        ],
        compiler_params=_SC_PARAMS,
    )


def _deg_of(pd_ref):
    return jnp.maximum(pd_ref[0][:, 0:1] + pd_ref[1][:, 0:1], 1.0)


def _mm0_body(x_ref, p_ref, pd_ref, ws_ref, wn_ref, b_ref, h_ref, *hc_refs):
    hn = jnp.concatenate(
        [p_ref[0, 0] + p_ref[1, 0], p_ref[0, 1] + p_ref[1, 1]],
        axis=1) / _deg_of(pd_ref)
    h = (jnp.dot(x_ref[...], ws_ref[...], preferred_element_type=jnp.float32)
         + jnp.dot(hn, wn_ref[...], preferred_element_type=jnp.float32)
         + b_ref[...])
    h = jnp.maximum(h, 0.0)
    h_ref[...] = h
    for j, hc in enumerate(hc_refs):
        hc[...] = h[:, CA * j:CA * (j + 1)]


def _mm1_body(h0_ref, p1_ref, pd_ref, ws_ref, wn_ref, b_ref, wn2_ref,
              h1_ref, y2_ref):
    hn = jnp.concatenate(
        [p1_ref[0, j] + p1_ref[1, j] for j in range(4)],
        axis=1) / _deg_of(pd_ref)
    h1 = (jnp.dot(h0_ref[...], ws_ref[...], preferred_element_type=jnp.float32)
          + jnp.dot(hn, wn_ref[...], preferred_element_type=jnp.float32)
          + b_ref[...])
    h1 = jnp.maximum(h1, 0.0)
    h1_ref[...] = h1
    y2_ref[...] = jnp.dot(h1, wn2_ref[...], preferred_element_type=jnp.float32)


def _mm2_body(h1_ref, p2_ref, pd_ref, ws_ref, b_ref, o_ref):
    hn = (p2_ref[0, 0] + p2_ref[1, 0]) / _deg_of(pd_ref)
    o_ref[...] = (jnp.dot(h1_ref[...], ws_ref[...],
                          preferred_element_type=jnp.float32)
                  + hn + b_ref[...])


BN = 1000
_G = N // BN


def _full(shape):
    return pl.BlockSpec(shape, lambda i: tuple(0 for _ in shape))


def _rows(shape):
    nd = len(shape)
    return pl.BlockSpec(shape, lambda i, nd=nd: tuple(
        i if d == nd - 2 else 0 for d in range(nd)))


def kernel(inputs, edge_index, W_self0, W_neigh0, b0, W_self1, W_neigh1, b1,
           W_self2, W_neigh2, b2):
    x = inputs
    edges = edge_index.reshape(2, NW * ITERS, B)

    pd = _make_sc_deg()(edges, jnp.ones((B, CD), jnp.float32),
                        jnp.zeros((N, CD), jnp.float32))

    x0 = lax.slice(x, (0, 0), (N, CA))
    x1 = lax.slice(x, (0, CA), (N, 2 * CA))
    p0 = _sc_agg([x0, x1], edges)

    mm0_out = pl.pallas_call(
        _mm0_body,
        grid=(_G,),
        in_specs=[
            _rows((BN, D_IN)),
            _rows((2, 2, BN, CA)),
            _rows((2, BN, CD)),
            _full((D_IN, D_H)),
            _full((D_IN, D_H)),
            _full((1, D_H)),
        ],
        out_specs=[_rows((BN, D_H))] + [_rows((BN, CA))] * 4,
        out_shape=([jax.ShapeDtypeStruct((N, D_H), jnp.float32)]
                   + [jax.ShapeDtypeStruct((N, CA), jnp.float32)] * 4),
    )(x, p0, pd, W_self0, W_neigh0, b0.reshape(1, -1))
    h0, h0c = mm0_out[0], mm0_out[1:]

    p1 = _sc_agg(list(h0c), edges)

    h1, y2 = pl.pallas_call(
        _mm1_body,
        grid=(_G,),
        in_specs=[
            _rows((BN, D_H)),
            _rows((2, 4, BN, CA)),
            _rows((2, BN, CD)),
            _full((D_H, D_H)),
            _full((D_H, D_H)),
            _full((1, D_H)),
            _full((D_H, D_OUT)),
        ],
        out_specs=[_rows((BN, D_H)), _rows((BN, D_OUT))],
        out_shape=[jax.ShapeDtypeStruct((N, D_H), jnp.float32),
                   jax.ShapeDtypeStruct((N, D_OUT), jnp.float32)],
    )(h0, p1, pd, W_self1, W_neigh1, b1.reshape(1, -1), W_neigh2)

    p2 = _sc_agg([y2], edges)

    out = pl.pallas_call(
        _mm2_body,
        grid=(_G,),
        in_specs=[
            _rows((BN, D_H)),
            _rows((2, 1, BN, CA)),
            _rows((2, BN, CD)),
            _full((D_H, D_OUT)),
            _full((1, D_OUT)),
        ],
        out_specs=_rows((BN, D_OUT)),
        out_shape=jax.ShapeDtypeStruct((N, D_OUT), jnp.float32),
    )(h1, p2, pd, W_self2, b2.reshape(1, -1))

    return (out, h0, h1)


# speedup vs baseline: 1.5077x; 1.1285x over previous
import functools

import jax
import jax.numpy as jnp
from jax import lax
from jax.experimental import pallas as pl
from jax.experimental.pallas import tpu as pltpu
from jax.experimental.pallas import tpu_sc as plsc

N = 10000
E = 320000
D_IN = 128
D_H = 256
D_OUT = 64

NC = 2
NS = 16
NW = NC * NS

B = 80
ITERS = E // (NW * B)
NR = 5
NG = ITERS // NR
RPT = N // NS

CA = 64
CD = 16

_SC_PARAMS = pltpu.CompilerParams(use_tc_tiling_on_sc=False)
_MESH = dict(core_axis_name="c", subcore_axis_name="s")


def _sc_agg_body(K, chunk_cols, *args):
    xs = args[:K]
    edges, zeros, out = args[K:K + 3]
    scratch = args[K + 3:]
    e_srcv, e_dstv = scratch[0], scratch[1]
    rows = scratch[2:2 + NR]
    aggS, xS = scratch[2 + NR], scratch[3 + NR]
    sg = scratch[4 + NR:4 + 2 * NR]
    ss = scratch[4 + 2 * NR:4 + 3 * NR]
    c = lax.axis_index("c")
    s = lax.axis_index("s")
    wid = c * NS + s
    r0 = s * RPT
    pltpu.sync_copy(edges.at[0, pl.ds(wid * ITERS, ITERS)], e_srcv)
    pltpu.sync_copy(edges.at[1, pl.ds(wid * ITERS, ITERS)], e_dstv)
    for k in range(K):
        xk = xs[k]
        co = chunk_cols[k]
        pltpu.sync_copy(zeros.at[pl.ds(r0, RPT)], aggS.at[pl.ds(r0, RPT)])
        pltpu.sync_copy(xk.at[pl.ds(r0, RPT), pl.ds(co, CA)],
                        xS.at[pl.ds(r0, RPT)])
        plsc.subcore_barrier()

        def fire_g(j, b, xk=xk):
            pltpu.async_copy(xS.at[e_srcv.at[j]], rows[b], sg[b])

        def wait_g(j, b, xk=xk):
            pltpu.make_async_copy(xS.at[e_srcv.at[j]], rows[b], sg[b]).wait()

        def fire_s(j, b):
            pltpu.async_copy(rows[b], aggS.at[e_dstv.at[j]], ss[b], add=True)

        def wait_s(j, b):
            pltpu.make_async_copy(rows[b], aggS.at[e_dstv.at[j]], ss[b]).wait()

        fire_g(0, 0)
        fire_g(1, 1)
        fire_g(2, 2)

        def it(i, carry):
            for r in range(NR):
                j = NR * i + r
                b3 = (r + 3) % NR
                if r <= 1:
                    @pl.when(i >= 1)
                    def _():
                        wait_s(j - 2, b3)

                    fire_g(j + 3, b3)
                else:
                    wait_s(j - 2, b3)

                    @pl.when(i < NG - 1)
                    def _():
                        fire_g(j + 3, b3)
                wait_g(j, r)
                fire_s(j, r)
            return carry

        lax.fori_loop(0, NG, it, 0)
        wait_s(ITERS - 2, (ITERS - 2) % NR)
        wait_s(ITERS - 1, (ITERS - 1) % NR)
        plsc.subcore_barrier()
        pltpu.sync_copy(aggS.at[pl.ds(r0, RPT)],
                        out.at[c, k // 2, pl.ds(r0, RPT),
                               pl.ds((k % 2) * CA, CA)])
        if k + 1 < K:
            plsc.subcore_barrier()


@functools.cache
def _make_sc_agg(K, chunk_cols):
    return pl.kernel(
        functools.partial(_sc_agg_body, K, chunk_cols),
        out_type=jax.ShapeDtypeStruct((NC, (K + 1) // 2, N, 2 * CA),
                                      jnp.float32),
        mesh=plsc.VectorSubcoreMesh(**_MESH),
        scratch_types=(
            [pltpu.VMEM((ITERS, B), jnp.int32)] * 2
            + [pltpu.VMEM((B, CA), jnp.float32)] * NR
            + [pltpu.VMEM_SHARED((N, CA), jnp.float32)] * 2
            + [pltpu.SemaphoreType.DMA] * (2 * NR)
        ),
        compiler_params=_SC_PARAMS,
    )


def _sc_agg(xchunks, chunk_cols, edges):
    zeros = jnp.zeros((N, CA), jnp.float32)
    return _make_sc_agg(len(xchunks), tuple(chunk_cols))(*xchunks, edges, zeros)


def _sc_deg_body(edges, ones_h, zeros_d, out_d, e_dstv, ones_v, degS, ss0, ss1):
    c = lax.axis_index("c")
    s = lax.axis_index("s")
    wid = c * NS + s
    r0 = s * RPT
    pltpu.sync_copy(edges.at[1, pl.ds(wid * ITERS, ITERS)], e_dstv)
    pltpu.sync_copy(ones_h, ones_v)
    pltpu.sync_copy(zeros_d.at[pl.ds(r0, RPT)], degS.at[pl.ds(r0, RPT)])
    plsc.subcore_barrier()

    def fire_s(j, sem):
        pltpu.async_copy(ones_v, degS.at[e_dstv.at[j]], sem, add=True)

    def wait_s(j, sem):
        pltpu.make_async_copy(ones_v, degS.at[e_dstv.at[j]], sem).wait()

    def it(i, carry):
        j = NR * i
        for r in range(NR):
            fire_s(j + r, ss0 if r % 2 == 0 else ss1)
        for r in range(NR):
            wait_s(j + r, ss0 if r % 2 == 0 else ss1)
        return carry

    lax.fori_loop(0, NG, it, 0)
    plsc.subcore_barrier()
    pltpu.sync_copy(degS.at[pl.ds(r0, RPT)], out_d.at[c, pl.ds(r0, RPT)])


@functools.cache
def _make_sc_deg():
    return pl.kernel(
        _sc_deg_body,
        out_type=jax.ShapeDtypeStruct((NC, N, CD), jnp.float32),
        mesh=plsc.VectorSubcoreMesh(**_MESH),
        scratch_types=[
            pltpu.VMEM((ITERS, B), jnp.int32),
            pltpu.VMEM((B, CD), jnp.float32),
            pltpu.VMEM_SHARED((N, CD), jnp.float32),
            pltpu.SemaphoreType.DMA,
            pltpu.SemaphoreType.DMA,
        ],
        compiler_params=_SC_PARAMS,
    )


def _deg_of(pd_ref):
    return jnp.maximum(pd_ref[0][:, 0:1] + pd_ref[1][:, 0:1], 1.0)


def _mm0_body(x_ref, p_ref, pd_ref, ws_ref, wn_ref, b_ref, h_ref):
    hn = (p_ref[0, 0] + p_ref[1, 0]) / _deg_of(pd_ref)
    h = (jnp.dot(x_ref[...], ws_ref[...], preferred_element_type=jnp.float32)
         + jnp.dot(hn, wn_ref[...], preferred_element_type=jnp.float32)
         + b_ref[...])
    h_ref[...] = jnp.maximum(h, 0.0)


def _mm1_body(h0_ref, p1_ref, pd_ref, ws_ref, wn_ref, b_ref, wn2_ref,
              h1_ref, y2_ref):
    hn = jnp.concatenate(
        [p1_ref[0, j] + p1_ref[1, j] for j in range(2)],
        axis=1) / _deg_of(pd_ref)
    h1 = (jnp.dot(h0_ref[...], ws_ref[...], preferred_element_type=jnp.float32)
          + jnp.dot(hn, wn_ref[...], preferred_element_type=jnp.float32)
          + b_ref[...])
    h1 = jnp.maximum(h1, 0.0)
    h1_ref[...] = h1
    y2_ref[...] = jnp.dot(h1, wn2_ref[...], preferred_element_type=jnp.float32)


def _mm2_body(h1_ref, p2_ref, pd_ref, ws_ref, b_ref, o_ref):
    hn = (p2_ref[0, 0][:, :CA] + p2_ref[1, 0][:, :CA]) / _deg_of(pd_ref)
    o_ref[...] = (jnp.dot(h1_ref[...], ws_ref[...],
                          preferred_element_type=jnp.float32)
                  + hn + b_ref[...])


BN = 1000
_G = N // BN


def _full(shape):
    return pl.BlockSpec(shape, lambda i: tuple(0 for _ in shape))


def _rows(shape):
    nd = len(shape)
    return pl.BlockSpec(shape, lambda i, nd=nd: tuple(
        i if d == nd - 2 else 0 for d in range(nd)))


def kernel(inputs, edge_index, W_self0, W_neigh0, b0, W_self1, W_neigh1, b1,
           W_self2, W_neigh2, b2):
    x = inputs
    edges = edge_index.reshape(2, NW * ITERS, B)

    pd = _make_sc_deg()(edges, jnp.ones((B, CD), jnp.float32),
                        jnp.zeros((N, CD), jnp.float32))

    p0 = _sc_agg([x, x], (0, CA), edges)

    h0 = pl.pallas_call(
        _mm0_body,
        grid=(_G,),
        in_specs=[
            _rows((BN, D_IN)),
            _rows((2, 1, BN, 2 * CA)),
            _rows((2, BN, CD)),
            _full((D_IN, D_H)),
            _full((D_IN, D_H)),
            _full((1, D_H)),
        ],
        out_specs=_rows((BN, D_H)),
        out_shape=jax.ShapeDtypeStruct((N, D_H), jnp.float32),
    )(x, p0, pd, W_self0, W_neigh0, b0.reshape(1, -1))

    p1 = _sc_agg([h0] * 4, (0, CA, 2 * CA, 3 * CA), edges)

    h1, y2 = pl.pallas_call(
        _mm1_body,
        grid=(_G,),
        in_specs=[
            _rows((BN, D_H)),
            _rows((2, 2, BN, 2 * CA)),
            _rows((2, BN, CD)),
            _full((D_H, D_H)),
            _full((D_H, D_H)),
            _full((1, D_H)),
            _full((D_H, D_OUT)),
        ],
        out_specs=[_rows((BN, D_H)), _rows((BN, D_OUT))],
        out_shape=[jax.ShapeDtypeStruct((N, D_H), jnp.float32),
                   jax.ShapeDtypeStruct((N, D_OUT), jnp.float32)],
    )(h0, p1, pd, W_self1, W_neigh1, b1.reshape(1, -1), W_neigh2)

    p2 = _sc_agg([y2], (0,), edges)

    out = pl.pallas_call(
        _mm2_body,
        grid=(_G,),
        in_specs=[
            _rows((BN, D_H)),
            _rows((2, 1, BN, 2 * CA)),
            _rows((2, BN, CD)),
            _full((D_H, D_OUT)),
            _full((1, D_OUT)),
        ],
        out_specs=_rows((BN, D_OUT)),
        out_shape=jax.ShapeDtypeStruct((N, D_OUT), jnp.float32),
    )(h1, p2, pd, W_self2, b2.reshape(1, -1))

    return (out, h0, h1)


# speedup vs baseline: 1.5166x; 1.0059x over previous
import functools

import jax
import jax.numpy as jnp
from jax import lax
from jax.experimental import pallas as pl
from jax.experimental.pallas import tpu as pltpu
from jax.experimental.pallas import tpu_sc as plsc

N = 10000
E = 320000
D_IN = 128
D_H = 256
D_OUT = 64

NC = 2
NS = 16
NW = NC * NS

B = 80
ITERS = E // (NW * B)
NR = 5
NG = ITERS // NR
RPT = N // NS

CA = 64
CD = 16

_SC_PARAMS = pltpu.CompilerParams(use_tc_tiling_on_sc=False)
_MESH = dict(core_axis_name="c", subcore_axis_name="s")


def _sc_agg_body(K, chunk_cols, *args):
    xs = args[:K]
    edges, zeros, out = args[K:K + 3]
    scratch = args[K + 3:]
    e_srcv, e_dstv = scratch[0], scratch[1]
    rows = scratch[2:2 + NR]
    aggS, xS = scratch[2 + NR], scratch[3 + NR]
    sg = scratch[4 + NR:4 + 2 * NR]
    ss = scratch[4 + 2 * NR:4 + 3 * NR]
    c = lax.axis_index("c")
    s = lax.axis_index("s")
    wid = c * NS + s
    r0 = s * RPT
    pltpu.sync_copy(edges.at[0, pl.ds(wid * ITERS, ITERS)], e_srcv)
    pltpu.sync_copy(edges.at[1, pl.ds(wid * ITERS, ITERS)], e_dstv)
    for k in range(K):
        xk = xs[k]
        co = chunk_cols[k]
        pltpu.sync_copy(zeros.at[pl.ds(r0, RPT)], aggS.at[pl.ds(r0, RPT)])
        pltpu.sync_copy(xk.at[pl.ds(r0, RPT), pl.ds(co, CA)],
                        xS.at[pl.ds(r0, RPT)])
        plsc.subcore_barrier()

        def fire_g(j, b, xk=xk):
            pltpu.async_copy(xS.at[e_srcv.at[j]], rows[b], sg[b])

        def wait_g(j, b, xk=xk):
            pltpu.make_async_copy(xS.at[e_srcv.at[j]], rows[b], sg[b]).wait()

        def fire_s(j, b):
            pltpu.async_copy(rows[b], aggS.at[e_dstv.at[j]], ss[b], add=True)

        def wait_s(j, b):
            pltpu.make_async_copy(rows[b], aggS.at[e_dstv.at[j]], ss[b]).wait()

        fire_g(0, 0)
        fire_g(1, 1)
        fire_g(2, 2)

        def it(i, carry):
            for r in range(NR):
                j = NR * i + r
                b3 = (r + 3) % NR
                if r <= 1:
                    @pl.when(i >= 1)
                    def _():
                        wait_s(j - 2, b3)

                    fire_g(j + 3, b3)
                else:
                    wait_s(j - 2, b3)

                    @pl.when(i < NG - 1)
                    def _():
                        fire_g(j + 3, b3)
                wait_g(j, r)
                fire_s(j, r)
            return carry

        lax.fori_loop(0, NG, it, 0)
        wait_s(ITERS - 2, (ITERS - 2) % NR)
        wait_s(ITERS - 1, (ITERS - 1) % NR)
        plsc.subcore_barrier()
        pltpu.sync_copy(aggS.at[pl.ds(r0, RPT)],
                        out.at[c, k // 2, pl.ds(r0, RPT),
                               pl.ds((k % 2) * CA, CA)])
        if k + 1 < K:
            plsc.subcore_barrier()


@functools.cache
def _make_sc_agg(K, chunk_cols):
    return pl.kernel(
        functools.partial(_sc_agg_body, K, chunk_cols),
        out_type=jax.ShapeDtypeStruct((NC, (K + 1) // 2, N, 2 * CA),
                                      jnp.float32),
        mesh=plsc.VectorSubcoreMesh(**_MESH),
        scratch_types=(
            [pltpu.VMEM((ITERS, B), jnp.int32)] * 2
            + [pltpu.VMEM((B, CA), jnp.float32)] * NR
            + [pltpu.VMEM_SHARED((N, CA), jnp.float32)] * 2
            + [pltpu.SemaphoreType.DMA] * (2 * NR)
        ),
        compiler_params=_SC_PARAMS,
    )


def _sc_agg(xchunks, chunk_cols, edges):
    zeros = jnp.zeros((N, CA), jnp.float32)
    return _make_sc_agg(len(xchunks), tuple(chunk_cols))(*xchunks, edges, zeros)


def _sc_deg_body(edges, ones_h, zeros_d, out_d, e_dstv, ones_v, degS, ss0, ss1):
    c = lax.axis_index("c")
    s = lax.axis_index("s")
    wid = c * NS + s
    r0 = s * RPT
    pltpu.sync_copy(edges.at[1, pl.ds(wid * ITERS, ITERS)], e_dstv)
    pltpu.sync_copy(ones_h, ones_v)
    pltpu.sync_copy(zeros_d.at[pl.ds(r0, RPT)], degS.at[pl.ds(r0, RPT)])
    plsc.subcore_barrier()

    def fire_s(j, sem):
        pltpu.async_copy(ones_v, degS.at[e_dstv.at[j]], sem, add=True)

    def wait_s(j, sem):
        pltpu.make_async_copy(ones_v, degS.at[e_dstv.at[j]], sem).wait()

    def it(i, carry):
        j = NR * i
        for r in range(NR):
            fire_s(j + r, ss0 if r % 2 == 0 else ss1)
        for r in range(NR):
            wait_s(j + r, ss0 if r % 2 == 0 else ss1)
        return carry

    lax.fori_loop(0, NG, it, 0)
    plsc.subcore_barrier()
    pltpu.sync_copy(degS.at[pl.ds(r0, RPT)], out_d.at[c, pl.ds(r0, RPT)])


@functools.cache
def _make_sc_deg():
    return pl.kernel(
        _sc_deg_body,
        out_type=jax.ShapeDtypeStruct((NC, N, CD), jnp.float32),
        mesh=plsc.VectorSubcoreMesh(**_MESH),
        scratch_types=[
            pltpu.VMEM((ITERS, B), jnp.int32),
            pltpu.VMEM((B, CD), jnp.float32),
            pltpu.VMEM_SHARED((N, CD), jnp.float32),
            pltpu.SemaphoreType.DMA,
            pltpu.SemaphoreType.DMA,
        ],
        compiler_params=_SC_PARAMS,
    )


def _deg_of(pd_ref):
    return jnp.maximum(pd_ref[0][:, 0:1] + pd_ref[1][:, 0:1], 1.0)


def _mm_self_body(x_ref, ws_ref, b_ref, t_ref):
    t_ref[...] = (jnp.dot(x_ref[...], ws_ref[...],
                          preferred_element_type=jnp.float32) + b_ref[...])


def _mm0_body(t_ref, p_ref, pd_ref, wn_ref, h_ref):
    hn = (p_ref[0, 0] + p_ref[1, 0]) / _deg_of(pd_ref)
    h = t_ref[...] + jnp.dot(hn, wn_ref[...],
                             preferred_element_type=jnp.float32)
    h_ref[...] = jnp.maximum(h, 0.0)


def _mm1_body(t_ref, p1_ref, pd_ref, wn_ref, wn2_ref, h1_ref, y2_ref):
    hn = jnp.concatenate(
        [p1_ref[0, j] + p1_ref[1, j] for j in range(2)],
        axis=1) / _deg_of(pd_ref)
    h1 = t_ref[...] + jnp.dot(hn, wn_ref[...],
                              preferred_element_type=jnp.float32)
    h1 = jnp.maximum(h1, 0.0)
    h1_ref[...] = h1
    y2_ref[:, :D_OUT] = jnp.dot(h1, wn2_ref[...],
                                preferred_element_type=jnp.float32)


def _mm2_body(t_ref, p2_ref, pd_ref, o_ref):
    hn = (p2_ref[0, 0][:, :CA] + p2_ref[1, 0][:, :CA]) / _deg_of(pd_ref)
    o_ref[...] = t_ref[...] + hn


BN = 1000
_G = N // BN


def _full(shape):
    return pl.BlockSpec(shape, lambda i: tuple(0 for _ in shape))


def _rows(shape):
    nd = len(shape)
    return pl.BlockSpec(shape, lambda i, nd=nd: tuple(
        i if d == nd - 2 else 0 for d in range(nd)))


def kernel(inputs, edge_index, W_self0, W_neigh0, b0, W_self1, W_neigh1, b1,
           W_self2, W_neigh2, b2):
    x = inputs
    edges = edge_index.reshape(2, NW * ITERS, B)

    pd = _make_sc_deg()(edges, jnp.ones((B, CD), jnp.float32),
                        jnp.zeros((N, CD), jnp.float32))

    def mm_self(xin, ws, b, dout):
        din = xin.shape[1]
        return pl.pallas_call(
            _mm_self_body,
            grid=(_G,),
            in_specs=[_rows((BN, din)), _full((din, dout)), _full((1, dout))],
            out_specs=_rows((BN, dout)),
            out_shape=jax.ShapeDtypeStruct((N, dout), jnp.float32),
        )(xin, ws, b.reshape(1, -1))

    p0 = _sc_agg([x, x], (0, CA), edges)
    t0 = mm_self(x, W_self0, b0, D_H)

    h0 = pl.pallas_call(
        _mm0_body,
        grid=(_G,),
        in_specs=[
            _rows((BN, D_H)),
            _rows((2, 1, BN, 2 * CA)),
            _rows((2, BN, CD)),
            _full((D_IN, D_H)),
        ],
        out_specs=_rows((BN, D_H)),
        out_shape=jax.ShapeDtypeStruct((N, D_H), jnp.float32),
    )(t0, p0, pd, W_neigh0)

    p1 = _sc_agg([h0] * 4, (0, CA, 2 * CA, 3 * CA), edges)
    t1 = mm_self(h0, W_self1, b1, D_H)

    h1, y2 = pl.pallas_call(
        _mm1_body,
        grid=(_G,),
        in_specs=[
            _rows((BN, D_H)),
            _rows((2, 2, BN, 2 * CA)),
            _rows((2, BN, CD)),
            _full((D_H, D_H)),
            _full((D_H, D_OUT)),
        ],
        out_specs=[_rows((BN, D_H)), _rows((BN, 2 * CA))],
        out_shape=[jax.ShapeDtypeStruct((N, D_H), jnp.float32),
                   jax.ShapeDtypeStruct((N, 2 * CA), jnp.float32)],
    )(t1, p1, pd, W_neigh1, W_neigh2)

    p2 = _sc_agg([y2], (0,), edges)
    t2 = mm_self(h1, W_self2, b2, D_OUT)

    out = pl.pallas_call(
        _mm2_body,
        grid=(_G,),
        in_specs=[
            _rows((BN, D_OUT)),
            _rows((2, 1, BN, 2 * CA)),
            _rows((2, BN, CD)),
        ],
        out_specs=_rows((BN, D_OUT)),
        out_shape=jax.ShapeDtypeStruct((N, D_OUT), jnp.float32),
    )(t2, p2, pd)

    return (out, h0, h1)
